# Initial kernel scaffold; baseline (speedup 1.0000x reference)
import jax
import jax.numpy as jnp
from jax.experimental import pallas as pl


def kernel(x):
    raise NotImplementedError("write your pallas kernel here")


# speedup vs baseline: 20.1995x; 20.1995x over previous
import jax
import jax.numpy as jnp
from jax.experimental import pallas as pl
from jax.experimental.pallas import tpu as pltpu

_H = _W = 224
_K = 16
_S = 8
_P = 4
_LH = _LW = 28
_C = 96
_B = 2


def _fold_slice_kernel(x_ref, o_ref):
    d = x_ref[0, 0].reshape(2, 8, 2, 8, _LH, _LW)
    acc = jnp.zeros((8, 8, _LH + 1, _LW + 1), jnp.float32)
    for ai in range(2):
        for aj in range(2):
            blk = d[ai, :, aj, :, :, :]
            acc = acc + jnp.pad(
                blk, ((0, 0), (0, 0), (ai, 1 - ai), (aj, 1 - aj)))
    pad = acc.transpose(2, 0, 3, 1).reshape(8 * (_LH + 1), 8 * (_LW + 1))
    o_ref[0, 0] = pad[_P:_P + _H, _P:_P + _W]


def kernel(x):
    b, qk, l = x.shape
    xr = x.reshape(b, _C, _K * _K, _LH, _LW)
    out = pl.pallas_call(
        _fold_slice_kernel,
        grid=(b, _C),
        in_specs=[
            pl.BlockSpec(
                (1, 1, _K * _K, _LH, _LW),
                lambda i, j: (i, j, 0, 0, 0),
            )
        ],
        out_specs=pl.BlockSpec(
            (1, 1, _H, _W),
            lambda i, j: (i, j, 0, 0),
        ),
        out_shape=jax.ShapeDtypeStruct((b, _C, _H, _W), jnp.float32),
    )(xr)
    return out


# speedup vs baseline: 23.1058x; 1.1439x over previous
import jax
import jax.numpy as jnp
from jax.experimental import pallas as pl
from jax.experimental.pallas import tpu as pltpu

_H = _W = 224
_K = 16
_S = 8
_P = 4
_LH = _LW = 28
_C = 96
_B = 2


def _fold_slice_kernel(x_ref, o_ref):
    d = x_ref[0].reshape(2, 8, 2, 8, _LH, _LW)
    acc = jnp.zeros((8, 8, _LH + 1, _LW + 1), jnp.float32)
    for ai in range(2):
        for aj in range(2):
            blk = d[ai, :, aj, :, :, :]
            acc = acc + jnp.pad(
                blk, ((0, 0), (0, 0), (ai, 1 - ai), (aj, 1 - aj)))
    pad = acc.transpose(2, 0, 3, 1).reshape(8 * (_LH + 1), 8 * (_LW + 1))
    o_ref[0, 0] = pad[_P:_P + _H, _P:_P + _W]


def kernel(x):
    b, qk, l = x.shape
    out = pl.pallas_call(
        _fold_slice_kernel,
        grid=(b, _C),
        in_specs=[
            pl.BlockSpec(
                (1, _K * _K, _LH * _LW),
                lambda i, j: (i, j, 0),
            )
        ],
        out_specs=pl.BlockSpec(
            (1, 1, _H, _W),
            lambda i, j: (i, j, 0, 0),
        ),
        out_shape=jax.ShapeDtypeStruct((b, _C, _H, _W), jnp.float32),
    )(x)
    return out


# speedup vs baseline: 37.3517x; 1.6166x over previous
import jax
import jax.numpy as jnp
from jax import lax
from jax.experimental import pallas as pl
from jax.experimental.pallas import tpu as pltpu
from jax.experimental.pallas import tpu_sc as plsc

_H = _W = 224
_LH = _LW = 28
_C = 96
_B = 2
_NSLICE = _B * _C
_NW = 32
_SPW = _NSLICE // _NW


def _sc_fold(x_hbm, out_hbm, buf_a, buf_b, out_buf):
    wid = lax.axis_index("s") * 2 + lax.axis_index("c")
    lanes = lax.iota(jnp.int32, 16)

    def slice_body(s, carry):
        sg = wid * _SPW + s
        b = sg // _C
        c = sg % _C

        def u_body(u, carry_u):
            q0 = c * 256 + u * 16
            pltpu.sync_copy(x_hbm.at[b, pl.ds(q0, 16), :], buf_a)
            pltpu.sync_copy(x_hbm.at[b, pl.ds(q0 + 128, 16), :], buf_b)
            m0 = jnp.where(u < 4, 1, 0)

            zeros16 = jnp.zeros((16,), jnp.float32)
            for g in range(14):
                x4 = lanes + jnp.full((16,), 16 * g + 4, jnp.int32)
                terms = []
                for aj in (0, 1):
                    jj = x4 % jnp.full((16,), 8, jnp.int32) + jnp.full(
                        (16,), 8 * aj, jnp.int32)
                    lwv = x4 // jnp.full((16,), 8, jnp.int32) - jnp.full(
                        (16,), aj, jnp.int32)
                    lane_ok = (lwv >= jnp.full((16,), 0, jnp.int32)) & (
                        lwv <= jnp.full((16,), _LW - 1, jnp.int32))
                    lwc = jnp.minimum(
                        jnp.maximum(lwv, jnp.full((16,), 0, jnp.int32)),
                        jnp.full((16,), _LW - 1, jnp.int32))
                    terms.append((jj, lane_ok, lwc))

                def m_body(m, carry_m):
                    acc = zeros16
                    for ai, buf in ((0, buf_a), (1, buf_b)):
                        lh = m - ai
                        lh_v = jnp.full((16,), lh, jnp.int32)
                        ai_ok = (lh_v >= jnp.full((16,), 0, jnp.int32)) & (
                            lh_v <= jnp.full((16,), _LH - 1, jnp.int32))
                        lhc_v = jnp.minimum(
                            jnp.maximum(lh_v, jnp.full((16,), 0, jnp.int32)),
                            jnp.full((16,), _LH - 1, jnp.int32),
                        ) * jnp.full((16,), _LW, jnp.int32)
                        for (jj, lane_ok, lwc) in terms:
                            col = lwc + lhc_v
                            val = plsc.load_gather(buf, [jj, col])
                            ok = lane_ok & ai_ok
                            acc = acc + jnp.where(ok, val, zeros16)
                    out_buf[m - m0, pl.ds(16 * g, 16)] = acc
                    return carry_m

                lax.fori_loop(m0, m0 + _LH, m_body, 0)

            w = (u + 4) % 8
            pltpu.sync_copy(out_buf, out_hbm.at[b, c, :, w, :])
            return carry_u

        lax.fori_loop(0, 8, u_body, 0)
        return carry

    lax.fori_loop(0, _SPW, slice_body, 0)


def kernel(x):
    out5 = pl.kernel(
        _sc_fold,
        out_type=jax.ShapeDtypeStruct((_B, _C, _LH, 8, _W), jnp.float32),
        mesh=plsc.VectorSubcoreMesh(core_axis_name="c", subcore_axis_name="s"),
        compiler_params=pltpu.CompilerParams(needs_layout_passes=False),
        scratch_types=[
            pltpu.VMEM((16, 784), jnp.float32),
            pltpu.VMEM((16, 784), jnp.float32),
            pltpu.VMEM((_LH, _W), jnp.float32),
        ],
    )(x)
    return out5.reshape(_B, _C, _H, _W)


# speedup vs baseline: 45.9270x; 1.2296x over previous
import jax
import jax.numpy as jnp
from jax import lax
from jax.experimental import pallas as pl
from jax.experimental.pallas import tpu as pltpu
from jax.experimental.pallas import tpu_sc as plsc

_H = _W = 224
_LH = _LW = 28
_C = 96
_B = 2
_NSLICE = _B * _C
_NW = 32
_SPW = _NSLICE // _NW


def _i16(v):
    return jnp.full((16,), v, jnp.int32)


def _sc_fold(x_hbm, out_hbm, buf_a0, buf_a1, buf_b0, buf_b1, obuf,
             in_sem0, in_sem1, out_sem):
    wid = lax.axis_index("s") * 2 + lax.axis_index("c")
    lanes = lax.iota(jnp.int32, 16)
    bufs_a = (buf_a0, buf_a1)
    bufs_b = (buf_b0, buf_b1)
    in_sems = (in_sem0, in_sem1)

    def src_a(b, c, u):
        return x_hbm.at[b, pl.ds(c * 256 + 16 * u, 16), :]

    def src_b(b, c, u):
        return x_hbm.at[b, pl.ds(c * 256 + 16 * u + 128, 16), :]

    def dst(b, c, u):
        return out_hbm.at[b, c, :, (u + 4) % 8, :]

    def g_vecs(g, aj):
        x4 = lanes + _i16(16 * g + 4)
        jj = x4 % _i16(8) + _i16(8 * aj)
        lwv = x4 // _i16(8) - _i16(aj)
        ok = (lwv >= _i16(0)) & (lwv <= _i16(_LW - 1))
        lwc = jnp.minimum(jnp.maximum(lwv, _i16(0)), _i16(_LW - 1))
        masked = (g == 0 and aj == 1) or (g == 13 and aj == 0)
        return jj, lwc, ok, masked

    sg0 = wid * _SPW
    b0 = sg0 // _C
    c0 = sg0 % _C
    pltpu.async_copy(src_a(b0, c0, 0), buf_a0, in_sem0)
    pltpu.async_copy(src_b(b0, c0, 0), buf_b0, in_sem0)

    def slice_body(s, carry):
        sg = wid * _SPW + s
        b = sg // _C
        c = sg % _C

        for u in range(8):
            par = u % 2
            buf_a, buf_b = bufs_a[par], bufs_b[par]
            pltpu.make_async_copy(src_a(b, c, u), buf_a, in_sems[par]).wait()
            pltpu.make_async_copy(src_b(b, c, u), buf_b, in_sems[par]).wait()
            if u < 7:
                pltpu.async_copy(src_a(b, c, u + 1), bufs_a[1 - par],
                                 in_sems[1 - par])
                pltpu.async_copy(src_b(b, c, u + 1), bufs_b[1 - par],
                                 in_sems[1 - par])
            else:
                @pl.when(s < _SPW - 1)
                def _():
                    sg2 = sg + 1
                    b2 = sg2 // _C
                    c2 = sg2 % _C
                    pltpu.async_copy(src_a(b2, c2, 0), bufs_a[1 - par],
                                     in_sems[1 - par])
                    pltpu.async_copy(src_b(b2, c2, 0), bufs_b[1 - par],
                                     in_sems[1 - par])
            if u == 0:
                @pl.when(s > 0)
                def _():
                    pltpu.make_async_copy(obuf, dst(b, c, u), out_sem).wait()
            else:
                pltpu.make_async_copy(obuf, dst(b, c, u), out_sem).wait()

            for g in range(14):
                jd0, lwc0, ok0, msk0 = g_vecs(g, 0)
                jd1, lwc1, ok1, msk1 = g_vecs(g, 1)
                zv = jnp.zeros((16,), jnp.float32)

                def pair(buf, cvec):
                    t0 = plsc.load_gather(buf, [jd0, lwc0 + cvec])
                    t1 = plsc.load_gather(buf, [jd1, lwc1 + cvec])
                    if msk0:
                        t0 = jnp.where(ok0, t0, zv)
                    if msk1:
                        t1 = jnp.where(ok1, t1, zv)
                    return t0 + t1

                def m_body(m, carry_m):
                    acc = pair(
                        buf_a, jnp.full((16,), m * _LW, jnp.int32)
                    ) + pair(
                        buf_b, jnp.full((16,), (m - 1) * _LW, jnp.int32))
                    r = m - 1 if u < 4 else m
                    obuf[r, pl.ds(16 * g, 16)] = acc
                    return carry_m

                lax.fori_loop(1, _LH, m_body, 0)

                if u < 4:
                    acc = pair(buf_b, _i16((_LH - 1) * _LW))
                    obuf[_LH - 1, pl.ds(16 * g, 16)] = acc
                else:
                    acc = pair(buf_a, _i16(0))
                    obuf[0, pl.ds(16 * g, 16)] = acc

            pltpu.async_copy(obuf, dst(b, c, u), out_sem)
        return carry

    lax.fori_loop(0, _SPW, slice_body, 0)

    pltpu.make_async_copy(obuf, dst(0, 0, 7), out_sem).wait()


def kernel(x):
    out5 = pl.kernel(
        _sc_fold,
        out_type=jax.ShapeDtypeStruct((_B, _C, _LH, 8, _W), jnp.float32),
        mesh=plsc.VectorSubcoreMesh(core_axis_name="c", subcore_axis_name="s"),
        compiler_params=pltpu.CompilerParams(needs_layout_passes=False),
        scratch_types=[
            pltpu.VMEM((16, 784), jnp.float32),
            pltpu.VMEM((16, 784), jnp.float32),
            pltpu.VMEM((16, 784), jnp.float32),
            pltpu.VMEM((16, 784), jnp.float32),
            pltpu.VMEM((_LH, _W), jnp.float32),
            pltpu.SemaphoreType.DMA,
            pltpu.SemaphoreType.DMA,
            pltpu.SemaphoreType.DMA,
        ],
    )(x)
    return out5.reshape(_B, _C, _H, _W)


# speedup vs baseline: 51.0339x; 1.1112x over previous
import jax
import jax.numpy as jnp
from jax import lax
from jax.experimental import pallas as pl
from jax.experimental.pallas import tpu as pltpu
from jax.experimental.pallas import tpu_sc as plsc

_H = _W = 224
_LH = _LW = 28
_C = 96
_B = 2
_NSLICE = _B * _C
_NW = 32
_SPW = _NSLICE // _NW


def _i16(v):
    return jnp.full((16,), v, jnp.int32)


def _sc_fold(x_hbm, out_hbm, buf_a0, buf_a1, buf_b0, buf_b1, obuf,
             in_sem0, in_sem1, out_sem):
    wid = lax.axis_index("s") * 2 + lax.axis_index("c")
    lanes = lax.iota(jnp.int32, 16)
    bufs_a = (buf_a0, buf_a1)
    bufs_b = (buf_b0, buf_b1)
    in_sems = (in_sem0, in_sem1)

    def src_a(b, c, u):
        return x_hbm.at[b, pl.ds(c * 256 + 16 * u, 16), :]

    def src_b(b, c, u):
        return x_hbm.at[b, pl.ds(c * 256 + 16 * u + 128, 16), :]

    def dst(b, c, u):
        return out_hbm.at[b, c, :, (u + 4) % 8, :]

    def g_vecs(g, aj):
        x4 = lanes + _i16(16 * g + 4)
        jj = x4 % _i16(8) + _i16(8 * aj)
        lwv = x4 // _i16(8) - _i16(aj)
        ok = (lwv >= _i16(0)) & (lwv <= _i16(_LW - 1))
        lwc = jnp.minimum(jnp.maximum(lwv, _i16(0)), _i16(_LW - 1))
        masked = (g == 0 and aj == 1) or (g == 13 and aj == 0)
        return jj, lwc, ok, masked

    sg0 = wid * _SPW
    b0 = sg0 // _C
    c0 = sg0 % _C
    pltpu.async_copy(src_a(b0, c0, 0), buf_a0, in_sem0)
    pltpu.async_copy(src_b(b0, c0, 0), buf_b0, in_sem0)

    def pair_body(s, up, upar):
        sg = wid * _SPW + s
        b = sg // _C
        c = sg % _C
        u = 2 * up + upar
        par = upar
        buf_a, buf_b = bufs_a[par], bufs_b[par]
        pltpu.make_async_copy(src_a(b, c, u), buf_a, in_sems[par]).wait()
        pltpu.make_async_copy(src_b(b, c, u), buf_b, in_sems[par]).wait()
        nu_raw = u + 1
        wrap = nu_raw > 7
        sg2 = jnp.where(wrap, sg + 1, sg)
        nu = jnp.where(wrap, 0, nu_raw)
        b2 = sg2 // _C
        c2 = sg2 % _C

        @pl.when(jnp.logical_or(jnp.logical_not(wrap), s < _SPW - 1))
        def _():
            pltpu.async_copy(src_a(b2, c2, nu), bufs_a[1 - par],
                             in_sems[1 - par])
            pltpu.async_copy(src_b(b2, c2, nu), bufs_b[1 - par],
                             in_sems[1 - par])

        @pl.when(jnp.logical_or(s > 0, u > 0))
        def _():
            pltpu.make_async_copy(obuf, dst(b, c, u), out_sem).wait()

        low = u < 4
        roff = jnp.where(low, 1, 0)

        for g in range(14):
            jd0, lwc0, ok0, msk0 = g_vecs(g, 0)
            jd1, lwc1, ok1, msk1 = g_vecs(g, 1)
            zv = jnp.zeros((16,), jnp.float32)

            def pair(buf, cvec):
                t0 = plsc.load_gather(buf, [jd0, lwc0 + cvec])
                t1 = plsc.load_gather(buf, [jd1, lwc1 + cvec])
                if msk0:
                    t0 = jnp.where(ok0, t0, zv)
                if msk1:
                    t1 = jnp.where(ok1, t1, zv)
                return t0 + t1

            @plsc.parallel_loop(1, _LH, unroll=4)
            def _(m):
                acc = pair(
                    buf_a, jnp.full((16,), m * _LW, jnp.int32)
                ) + pair(
                    buf_b, jnp.full((16,), (m - 1) * _LW, jnp.int32))
                obuf[m - roff, pl.ds(16 * g, 16)] = acc

            @pl.when(low)
            def _():
                acc = pair(buf_b, _i16((_LH - 1) * _LW))
                obuf[_LH - 1, pl.ds(16 * g, 16)] = acc

            @pl.when(jnp.logical_not(low))
            def _():
                acc = pair(buf_a, _i16(0))
                obuf[0, pl.ds(16 * g, 16)] = acc

        pltpu.async_copy(obuf, dst(b, c, u), out_sem)

    def chunk_body(k, carry):
        s = k // 4
        up = k % 4
        pair_body(s, up, 0)
        pair_body(s, up, 1)
        return carry

    lax.fori_loop(0, _SPW * 4, chunk_body, 0)

    pltpu.make_async_copy(obuf, dst(0, 0, 7), out_sem).wait()


def kernel(x):
    out5 = pl.kernel(
        _sc_fold,
        out_type=jax.ShapeDtypeStruct((_B, _C, _LH, 8, _W), jnp.float32),
        mesh=plsc.VectorSubcoreMesh(core_axis_name="c", subcore_axis_name="s"),
        compiler_params=pltpu.CompilerParams(needs_layout_passes=False),
        scratch_types=[
            pltpu.VMEM((16, 784), jnp.float32),
            pltpu.VMEM((16, 784), jnp.float32),
            pltpu.VMEM((16, 784), jnp.float32),
            pltpu.VMEM((16, 784), jnp.float32),
            pltpu.VMEM((_LH, _W), jnp.float32),
            pltpu.SemaphoreType.DMA,
            pltpu.SemaphoreType.DMA,
            pltpu.SemaphoreType.DMA,
        ],
    )(x)
    return out5.reshape(_B, _C, _H, _W)


# speedup vs baseline: 52.1622x; 1.0221x over previous
import jax
import jax.numpy as jnp
from jax import lax
from jax.experimental import pallas as pl
from jax.experimental.pallas import tpu as pltpu
from jax.experimental.pallas import tpu_sc as plsc

_H = _W = 224
_LH = _LW = 28
_C = 96
_B = 2
_NSLICE = _B * _C
_NW = 32
_SPW = _NSLICE // _NW


def _i16(v):
    return jnp.full((16,), v, jnp.int32)


def _sc_fold(x_hbm, out_hbm, buf_a0, buf_a1, buf_b0, buf_b1, obuf0, obuf1,
             in_sem0, in_sem1, out_sem0, out_sem1):
    wid = lax.axis_index("s") * 2 + lax.axis_index("c")
    lanes = lax.iota(jnp.int32, 16)
    bufs_a = (buf_a0, buf_a1)
    bufs_b = (buf_b0, buf_b1)
    obufs = (obuf0, obuf1)
    in_sems = (in_sem0, in_sem1)
    out_sems = (out_sem0, out_sem1)

    def src_a(b, c, u):
        return x_hbm.at[b, pl.ds(c * 256 + 16 * u, 16), :]

    def src_b(b, c, u):
        return x_hbm.at[b, pl.ds(c * 256 + 16 * u + 128, 16), :]

    def dst(b, c, u):
        return out_hbm.at[b, c, :, (u + 4) % 8, :]

    def g_vecs(g, aj):
        x4 = lanes + _i16(16 * g + 4)
        jj = x4 % _i16(8) + _i16(8 * aj)
        lwv = x4 // _i16(8) - _i16(aj)
        ok = (lwv >= _i16(0)) & (lwv <= _i16(_LW - 1))
        lwc = jnp.minimum(jnp.maximum(lwv, _i16(0)), _i16(_LW - 1))
        masked = (g == 0 and aj == 1) or (g == 13 and aj == 0)
        return jj, lwc, ok, masked

    sg0 = wid * _SPW
    b0 = sg0 // _C
    c0 = sg0 % _C
    pltpu.async_copy(src_a(b0, c0, 0), buf_a0, in_sem0)
    pltpu.async_copy(src_b(b0, c0, 0), buf_b0, in_sem0)

    def pair_body(s, up, upar):
        sg = wid * _SPW + s
        b = sg // _C
        c = sg % _C
        u = 2 * up + upar
        par = upar
        buf_a, buf_b = bufs_a[par], bufs_b[par]
        obuf = obufs[par]
        out_sem = out_sems[par]
        pltpu.make_async_copy(src_a(b, c, u), buf_a, in_sems[par]).wait()
        pltpu.make_async_copy(src_b(b, c, u), buf_b, in_sems[par]).wait()
        nu_raw = u + 1
        wrap = nu_raw > 7
        sg2 = jnp.where(wrap, sg + 1, sg)
        nu = jnp.where(wrap, 0, nu_raw)
        b2 = sg2 // _C
        c2 = sg2 % _C

        @pl.when(jnp.logical_or(jnp.logical_not(wrap), s < _SPW - 1))
        def _():
            pltpu.async_copy(src_a(b2, c2, nu), bufs_a[1 - par],
                             in_sems[1 - par])
            pltpu.async_copy(src_b(b2, c2, nu), bufs_b[1 - par],
                             in_sems[1 - par])

        @pl.when(jnp.logical_or(s > 0, up > 0))
        def _():
            pltpu.make_async_copy(obuf, dst(b, c, u), out_sem).wait()

        low = u < 4
        roff = jnp.where(low, 1, 0)

        for g in range(14):
            jd0, lwc0, ok0, msk0 = g_vecs(g, 0)
            jd1, lwc1, ok1, msk1 = g_vecs(g, 1)
            zv = jnp.zeros((16,), jnp.float32)

            def pair(buf, cvec):
                t0 = plsc.load_gather(buf, [jd0, lwc0 + cvec])
                t1 = plsc.load_gather(buf, [jd1, lwc1 + cvec])
                if msk0:
                    t0 = jnp.where(ok0, t0, zv)
                if msk1:
                    t1 = jnp.where(ok1, t1, zv)
                return t0 + t1

            @plsc.parallel_loop(1, _LH, unroll=4)
            def _(m):
                acc = pair(
                    buf_a, jnp.full((16,), m * _LW, jnp.int32)
                ) + pair(
                    buf_b, jnp.full((16,), (m - 1) * _LW, jnp.int32))
                obuf[m - roff, pl.ds(16 * g, 16)] = acc

            @pl.when(low)
            def _():
                acc = pair(buf_b, _i16((_LH - 1) * _LW))
                obuf[_LH - 1, pl.ds(16 * g, 16)] = acc

            @pl.when(jnp.logical_not(low))
            def _():
                acc = pair(buf_a, _i16(0))
                obuf[0, pl.ds(16 * g, 16)] = acc

        pltpu.async_copy(obuf, dst(b, c, u), out_sem)

    def chunk_body(k, carry):
        s = k // 4
        up = k % 4
        pair_body(s, up, 0)
        pair_body(s, up, 1)
        return carry

    lax.fori_loop(0, _SPW * 4, chunk_body, 0)

    pltpu.make_async_copy(obuf0, dst(0, 0, 6), out_sem0).wait()
    pltpu.make_async_copy(obuf1, dst(0, 0, 7), out_sem1).wait()


def kernel(x):
    out5 = pl.kernel(
        _sc_fold,
        out_type=jax.ShapeDtypeStruct((_B, _C, _LH, 8, _W), jnp.float32),
        mesh=plsc.VectorSubcoreMesh(core_axis_name="c", subcore_axis_name="s"),
        compiler_params=pltpu.CompilerParams(needs_layout_passes=False),
        scratch_types=[
            pltpu.VMEM((16, 784), jnp.float32),
            pltpu.VMEM((16, 784), jnp.float32),
            pltpu.VMEM((16, 784), jnp.float32),
            pltpu.VMEM((16, 784), jnp.float32),
            pltpu.VMEM((_LH, _W), jnp.float32),
            pltpu.VMEM((_LH, _W), jnp.float32),
            pltpu.SemaphoreType.DMA,
            pltpu.SemaphoreType.DMA,
            pltpu.SemaphoreType.DMA,
            pltpu.SemaphoreType.DMA,
        ],
    )(x)
    return out5.reshape(_B, _C, _H, _W)
